# baseline (speedup 1.0000x reference)
import functools

import jax
import jax.numpy as jnp
from jax.experimental import pallas as pl
from jax.experimental.pallas import tpu as pltpu

_N = 16384
_D = 768
_BLK = 8192
_GRID = _N // _BLK


def _body(mask_ref, docs_ref, q_ref, out_ref, acc_ref, cnt_ref):
    i = pl.program_id(0)

    @pl.when(i == 0)
    def _init():
        acc_ref[...] = jnp.zeros_like(acc_ref)
        cnt_ref[0] = 0.0

    m = mask_ref[0]
    acc_ref[...] += jnp.dot(m, docs_ref[...], preferred_element_type=jnp.float32)
    cnt_ref[0] += jnp.sum(m)

    @pl.when(i == _GRID - 1)
    def _finalize():
        cnt = cnt_ref[0]
        mean = acc_ref[...] / jnp.maximum(cnt, 1.0)
        wa = (q_ref[...] + mean) * 0.5
        norm = jnp.maximum(jnp.sqrt(jnp.sum(wa * wa)), 1e-12)
        out_ref[...] = jnp.where(cnt == 0.0, q_ref[...], wa / norm)


@functools.partial(jax.jit, static_argnames=())
def kernel(question_embedding, document_embeddings, mask):
    maskf = mask.astype(jnp.float32).reshape(_GRID, 1, _BLK)
    q = question_embedding.reshape(1, _D)
    out = pl.pallas_call(
        _body,
        grid=(_GRID,),
        in_specs=[
            pl.BlockSpec((1, 1, _BLK), lambda i: (i, 0, 0)),
            pl.BlockSpec((_BLK, _D), lambda i: (i, 0)),
            pl.BlockSpec((1, _D), lambda i: (0, 0)),
        ],
        out_specs=pl.BlockSpec((1, _D), lambda i: (0, 0)),
        out_shape=jax.ShapeDtypeStruct((1, _D), jnp.float32),
        scratch_shapes=[
            pltpu.VMEM((1, _D), jnp.float32),
            pltpu.SMEM((1,), jnp.float32),
        ],
    )(maskf, document_embeddings, q)
    return out.reshape(_D)


# speedup vs baseline: 1.0637x; 1.0637x over previous
import functools

import jax
import jax.numpy as jnp
from jax.experimental import pallas as pl
from jax.experimental.pallas import tpu as pltpu

_N = 16384
_D = 768
_BLK = 2048
_GRID = _N // _BLK


def _body(mask_ref, docs_ref, q_ref, out_ref, acc_ref, cnt_ref):
    i = pl.program_id(0)

    @pl.when(i == 0)
    def _init():
        acc_ref[...] = jnp.zeros_like(acc_ref)
        cnt_ref[0] = 0.0

    m = mask_ref[0]
    acc_ref[...] += jnp.dot(m, docs_ref[...], preferred_element_type=jnp.float32)
    cnt_ref[0] += jnp.sum(m)

    @pl.when(i == _GRID - 1)
    def _finalize():
        cnt = cnt_ref[0]
        mean = acc_ref[...] / jnp.maximum(cnt, 1.0)
        wa = (q_ref[...] + mean) * 0.5
        norm = jnp.maximum(jnp.sqrt(jnp.sum(wa * wa)), 1e-12)
        out_ref[...] = jnp.where(cnt == 0.0, q_ref[...], wa / norm)


@functools.partial(jax.jit, static_argnames=())
def kernel(question_embedding, document_embeddings, mask):
    maskf = mask.astype(jnp.float32).reshape(_GRID, 1, _BLK)
    q = question_embedding.reshape(1, _D)
    out = pl.pallas_call(
        _body,
        grid=(_GRID,),
        in_specs=[
            pl.BlockSpec((1, 1, _BLK), lambda i: (i, 0, 0)),
            pl.BlockSpec((_BLK, _D), lambda i: (i, 0)),
            pl.BlockSpec((1, _D), lambda i: (0, 0)),
        ],
        out_specs=pl.BlockSpec((1, _D), lambda i: (0, 0)),
        out_shape=jax.ShapeDtypeStruct((1, _D), jnp.float32),
        scratch_shapes=[
            pltpu.VMEM((1, _D), jnp.float32),
            pltpu.SMEM((1,), jnp.float32),
        ],
    )(maskf, document_embeddings, q)
    return out.reshape(_D)
